# baseline (speedup 1.0000x reference)
import functools
import jax
import jax.numpy as jnp
from jax import lax
from jax.experimental import pallas as pl
from jax.experimental.pallas import tpu as pltpu
from jax.experimental.pallas import tpu_sc as plsc

_B = 4
_S = 2048
_L = 16
_NC = 2
_NS = 16
_NW = _NC * _NS
_CH = 16
_TJ = 16
_NJB = _S // (_TJ * _L)


def _sc_rows(S_sc):
    WPB = _NW // _B
    RW = S_sc // WPB
    NCH = RW // _CH
    mesh = plsc.VectorSubcoreMesh(core_axis_name="c", subcore_axis_name="s")

    @functools.partial(
        pl.kernel,
        mesh=mesh,
        out_type=jax.ShapeDtypeStruct((_B, S_sc, _L), jnp.float32),
        scratch_types=[
            pltpu.VMEM((_S,), jnp.float32),
            pltpu.VMEM((_CH, _S), jnp.float32),
            pltpu.VMEM((_CH, _S), jnp.float32),
            pltpu.VMEM((RW, _L), jnp.float32),
            pltpu.SemaphoreType.DMA,
            pltpu.SemaphoreType.DMA,
        ],
    )
    def sc_kernel(inp_hbm, mask_hbm, out_hbm,
                  inp_v, buf0, buf1, part_v, sem0, sem1):
        wid = lax.axis_index("s") * _NC + lax.axis_index("c")
        b = wid // WPB
        i0 = (wid % WPB) * RW

        pltpu.sync_copy(inp_hbm.at[b], inp_v)

        def start(c, buf, sem):
            pltpu.make_async_copy(
                mask_hbm.at[b, pl.ds(i0 + c * _CH, _CH), :], buf, sem
            ).start()

        def wait(buf, sem):
            pltpu.make_async_copy(
                mask_hbm.at[b, pl.ds(i0, _CH), :], buf, sem
            ).wait()

        def compute(buf, c):
            for jb in range(_NJB):
                base = jb * _TJ * _L
                inp_regs = [inp_v[pl.ds(base + t * _L, _L)] for t in range(_TJ)]

                @plsc.parallel_loop(0, _CH, 1, unroll=2)
                def row_body(r, jb=jb, base=base, inp_regs=inp_regs):
                    na = 8
                    accs = [inp_regs[a] * buf[r, pl.ds(base + a * _L, _L)]
                            for a in range(na)]
                    for t in range(na, _TJ):
                        a = t % na
                        accs[a] = accs[a] + inp_regs[t] * buf[r, pl.ds(base + t * _L, _L)]
                    while len(accs) > 1:
                        accs = [accs[i] + accs[i + 1]
                                for i in range(0, len(accs), 2)]
                    if jb == 0:
                        part_v[c * _CH + r] = accs[0]
                    else:
                        plsc.addupdate(part_v.at[c * _CH + r], accs[0])

        start(0, buf0, sem0)

        def pair_body(p, _):
            c0 = 2 * p
            start(c0 + 1, buf1, sem1)
            wait(buf0, sem0)
            compute(buf0, c0)

            @pl.when(c0 + 2 < NCH)
            def _():
                start(c0 + 2, buf0, sem0)

            wait(buf1, sem1)
            compute(buf1, c0 + 1)
            return 0

        lax.fori_loop(0, NCH // 2, pair_body, 0)
        pltpu.sync_copy(part_v, out_hbm.at[b, pl.ds(i0, RW), :])

    return sc_kernel


def _finish_kernel(p_ref, o_ref):
    o_ref[...] = jnp.sum(p_ref[...], axis=-1)


def _tc_finish(partials, S_sc):
    TS = min(512, S_sc)
    return pl.pallas_call(
        _finish_kernel,
        grid=(S_sc // TS,),
        in_specs=[pl.BlockSpec((_B, TS, _L), lambda i: (0, i, 0))],
        out_specs=pl.BlockSpec((_B, TS), lambda i: (0, i)),
        out_shape=jax.ShapeDtypeStruct((_B, S_sc), jnp.float32),
    )(partials)


def _tc_matvec_kernel(inp_ref, mask_ref, out_ref):
    v = inp_ref[...]
    out_ref[...] = jnp.sum(mask_ref[...] * v[:, None, :], axis=-1)


def _tc_rows(S_sc, TS=512):
    n_rows = _S - S_sc
    off = S_sc // TS
    return pl.pallas_call(
        _tc_matvec_kernel,
        grid=(n_rows // TS,),
        in_specs=[
            pl.BlockSpec((_B, _S), lambda i: (0, 0)),
            pl.BlockSpec((_B, TS, _S), lambda i: (0, off + i, 0)),
        ],
        out_specs=pl.BlockSpec((_B, TS), lambda i: (0, i)),
        out_shape=jax.ShapeDtypeStruct((_B, n_rows), jnp.float32),
    )


def _tc_matvec2_kernel(inp_ref, mask_ref, out_ref):
    v = inp_ref[0, 0]
    out_ref[0, 0, 0, :] = jnp.sum(mask_ref[0] * v[None, :], axis=-1)


def _tc_flat(TS=512):
    NB = _S // TS
    return pl.pallas_call(
        _tc_matvec2_kernel,
        grid=(_B, NB),
        in_specs=[
            pl.BlockSpec((1, 1, _S), lambda b, i: (b, 0, 0)),
            pl.BlockSpec((1, TS, _S), lambda b, i: (b, i, 0)),
        ],
        out_specs=pl.BlockSpec((1, 1, 1, TS), lambda b, i: (b, i, 0, 0)),
        out_shape=jax.ShapeDtypeStruct((_B, NB, 1, TS), jnp.float32),
    )


_S_SC = 0


def _tc_dot_kernel(inp_ref, mask_ref, out_ref):
    m = mask_ref[...]
    v = inp_ref[...]
    out_ref[...] = lax.dot_general(
        m, v,
        dimension_numbers=(((2,), (1,)), ((0,), (0,))),
        preferred_element_type=jnp.float32,
    )


def _tc_dot(TS=128):
    return pl.pallas_call(
        _tc_dot_kernel,
        grid=(_S // TS,),
        in_specs=[
            pl.BlockSpec((_B, _S), lambda i: (0, 0)),
            pl.BlockSpec((_B, TS, _S), lambda i: (0, i, 0)),
        ],
        out_specs=pl.BlockSpec((_B, TS), lambda i: (0, i)),
        out_shape=jax.ShapeDtypeStruct((_B, _S), jnp.float32),
    )


def kernel(input, attention_mask):
    if _S_SC == 0:
        return _tc_dot()(input, attention_mask)
    partials = _sc_rows(_S_SC)(input, attention_mask)
    tc_out = _tc_rows(_S_SC)(input, attention_mask)
    sc_out = _tc_finish(partials, _S_SC)
    return jnp.concatenate([sc_out, tc_out], axis=1)


# speedup vs baseline: 1.0002x; 1.0002x over previous
import functools
import jax
import jax.numpy as jnp
from jax import lax
from jax.experimental import pallas as pl
from jax.experimental.pallas import tpu as pltpu
from jax.experimental.pallas import tpu_sc as plsc

_B = 4
_S = 2048
_L = 16
_NC = 2
_NS = 16
_NW = _NC * _NS
_CH = 16
_TJ = 16
_NJB = _S // (_TJ * _L)


def _sc_rows(S_sc):
    WPB = _NW // _B
    RW = S_sc // WPB
    NCH = RW // _CH
    mesh = plsc.VectorSubcoreMesh(core_axis_name="c", subcore_axis_name="s")

    @functools.partial(
        pl.kernel,
        mesh=mesh,
        out_type=jax.ShapeDtypeStruct((_B, S_sc, _L), jnp.float32),
        scratch_types=[
            pltpu.VMEM((_S,), jnp.float32),
            pltpu.VMEM((_CH, _S), jnp.float32),
            pltpu.VMEM((_CH, _S), jnp.float32),
            pltpu.VMEM((RW, _L), jnp.float32),
            pltpu.SemaphoreType.DMA,
            pltpu.SemaphoreType.DMA,
        ],
    )
    def sc_kernel(inp_hbm, mask_hbm, out_hbm,
                  inp_v, buf0, buf1, part_v, sem0, sem1):
        wid = lax.axis_index("s") * _NC + lax.axis_index("c")
        b = wid // WPB
        i0 = (wid % WPB) * RW

        pltpu.sync_copy(inp_hbm.at[b], inp_v)

        def start(c, buf, sem):
            pltpu.make_async_copy(
                mask_hbm.at[b, pl.ds(i0 + c * _CH, _CH), :], buf, sem
            ).start()

        def wait(buf, sem):
            pltpu.make_async_copy(
                mask_hbm.at[b, pl.ds(i0, _CH), :], buf, sem
            ).wait()

        def compute(buf, c):
            for jb in range(_NJB):
                base = jb * _TJ * _L
                inp_regs = [inp_v[pl.ds(base + t * _L, _L)] for t in range(_TJ)]

                @plsc.parallel_loop(0, _CH, 1, unroll=2)
                def row_body(r, jb=jb, base=base, inp_regs=inp_regs):
                    na = 8
                    accs = [inp_regs[a] * buf[r, pl.ds(base + a * _L, _L)]
                            for a in range(na)]
                    for t in range(na, _TJ):
                        a = t % na
                        accs[a] = accs[a] + inp_regs[t] * buf[r, pl.ds(base + t * _L, _L)]
                    while len(accs) > 1:
                        accs = [accs[i] + accs[i + 1]
                                for i in range(0, len(accs), 2)]
                    if jb == 0:
                        part_v[c * _CH + r] = accs[0]
                    else:
                        plsc.addupdate(part_v.at[c * _CH + r], accs[0])

        start(0, buf0, sem0)

        def pair_body(p, _):
            c0 = 2 * p
            start(c0 + 1, buf1, sem1)
            wait(buf0, sem0)
            compute(buf0, c0)

            @pl.when(c0 + 2 < NCH)
            def _():
                start(c0 + 2, buf0, sem0)

            wait(buf1, sem1)
            compute(buf1, c0 + 1)
            return 0

        lax.fori_loop(0, NCH // 2, pair_body, 0)
        pltpu.sync_copy(part_v, out_hbm.at[b, pl.ds(i0, RW), :])

    return sc_kernel


def _finish_kernel(p_ref, o_ref):
    o_ref[...] = jnp.sum(p_ref[...], axis=-1)


def _tc_finish(partials, S_sc):
    TS = min(512, S_sc)
    return pl.pallas_call(
        _finish_kernel,
        grid=(S_sc // TS,),
        in_specs=[pl.BlockSpec((_B, TS, _L), lambda i: (0, i, 0))],
        out_specs=pl.BlockSpec((_B, TS), lambda i: (0, i)),
        out_shape=jax.ShapeDtypeStruct((_B, S_sc), jnp.float32),
    )(partials)


def _tc_matvec_kernel(inp_ref, mask_ref, out_ref):
    v = inp_ref[...]
    out_ref[...] = jnp.sum(mask_ref[...] * v[:, None, :], axis=-1)


def _tc_rows(S_sc, TS=512):
    n_rows = _S - S_sc
    off = S_sc // TS
    return pl.pallas_call(
        _tc_matvec_kernel,
        grid=(n_rows // TS,),
        in_specs=[
            pl.BlockSpec((_B, _S), lambda i: (0, 0)),
            pl.BlockSpec((_B, TS, _S), lambda i: (0, off + i, 0)),
        ],
        out_specs=pl.BlockSpec((_B, TS), lambda i: (0, i)),
        out_shape=jax.ShapeDtypeStruct((_B, n_rows), jnp.float32),
    )


def _tc_matvec2_kernel(inp_ref, mask_ref, out_ref):
    v = inp_ref[0, 0]
    out_ref[0, 0, 0, :] = jnp.sum(mask_ref[0] * v[None, :], axis=-1)


def _tc_flat(TS=512):
    NB = _S // TS
    return pl.pallas_call(
        _tc_matvec2_kernel,
        grid=(_B, NB),
        in_specs=[
            pl.BlockSpec((1, 1, _S), lambda b, i: (b, 0, 0)),
            pl.BlockSpec((1, TS, _S), lambda b, i: (b, i, 0)),
        ],
        out_specs=pl.BlockSpec((1, 1, 1, TS), lambda b, i: (b, i, 0, 0)),
        out_shape=jax.ShapeDtypeStruct((_B, NB, 1, TS), jnp.float32),
    )


_S_SC = 0


def _tc_dot_kernel(inp_ref, mask_ref, out_ref):
    m = mask_ref[...]
    v = inp_ref[...]
    out_ref[...] = lax.dot_general(
        m, v,
        dimension_numbers=(((2,), (1,)), ((0,), (0,))),
        preferred_element_type=jnp.float32,
    )


def _tc_dot(TS=512):
    return pl.pallas_call(
        _tc_dot_kernel,
        grid=(_S // TS,),
        in_specs=[
            pl.BlockSpec((_B, _S), lambda i: (0, 0)),
            pl.BlockSpec((_B, TS, _S), lambda i: (0, i, 0)),
        ],
        out_specs=pl.BlockSpec((_B, TS), lambda i: (0, i)),
        out_shape=jax.ShapeDtypeStruct((_B, _S), jnp.float32),
    )


def kernel(input, attention_mask):
    if _S_SC == 0:
        return _tc_dot()(input, attention_mask)
    partials = _sc_rows(_S_SC)(input, attention_mask)
    tc_out = _tc_rows(_S_SC)(input, attention_mask)
    sc_out = _tc_finish(partials, _S_SC)
    return jnp.concatenate([sc_out, tc_out], axis=1)


# speedup vs baseline: 1.0357x; 1.0354x over previous
import jax
import jax.numpy as jnp
from jax import lax
from jax.experimental import pallas as pl

_TS = 256


def _matvec_kernel(inp_ref, mask_ref, out_ref):
    out_ref[...] = lax.dot_general(
        mask_ref[...],
        inp_ref[...],
        dimension_numbers=(((2,), (1,)), ((0,), (0,))),
        preferred_element_type=jnp.float32,
    )


def kernel(input, attention_mask):
    B, S = input.shape
    return pl.pallas_call(
        _matvec_kernel,
        grid=(S // _TS,),
        in_specs=[
            pl.BlockSpec((B, S), lambda i: (0, 0)),
            pl.BlockSpec((B, _TS, S), lambda i: (0, i, 0)),
        ],
        out_specs=pl.BlockSpec((B, _TS), lambda i: (0, i)),
        out_shape=jax.ShapeDtypeStruct((B, S), jnp.float32),
    )(input, attention_mask)


# speedup vs baseline: 1.0480x; 1.0119x over previous
import jax
import jax.numpy as jnp
from jax import lax
from jax.experimental import pallas as pl

_TS = 256


def _matvec_kernel(inp_ref, mask_ref, out_ref):
    i = pl.program_id(0)
    out_ref[:, pl.ds(i * _TS, _TS)] = lax.dot_general(
        mask_ref[...],
        inp_ref[...],
        dimension_numbers=(((2,), (1,)), ((0,), (0,))),
        preferred_element_type=jnp.float32,
    )


def kernel(input, attention_mask):
    B, S = input.shape
    return pl.pallas_call(
        _matvec_kernel,
        grid=(S // _TS,),
        in_specs=[
            pl.BlockSpec((B, S), lambda i: (0, 0)),
            pl.BlockSpec((B, _TS, S), lambda i: (0, i, 0)),
        ],
        out_specs=pl.BlockSpec((B, S), lambda i: (0, 0)),
        out_shape=jax.ShapeDtypeStruct((B, S), jnp.float32),
    )(input, attention_mask)
